# baseline (speedup 1.0000x reference)
import functools
import jax
import jax.numpy as jnp
from jax import lax
from jax.experimental import pallas as pl
from jax.experimental.pallas import tpu as pltpu
from jax.experimental.pallas import tpu_sc as plsc

V = 1000
N = 3
ROWS_PER_COPY = 2


def _sc_fill_kernel(rows_per_tile, pat_hbm, out_hbm, pat_v, sem):
    wid = lax.axis_index("s") * 2 + lax.axis_index("c")
    pltpu.sync_copy(pat_hbm, pat_v)

    base = wid * rows_per_tile
    n_copies = rows_per_tile // ROWS_PER_COPY
    for c in range(n_copies):
        pltpu.make_async_copy(
            pat_v,
            out_hbm.at[pl.ds(base + c * ROWS_PER_COPY, ROWS_PER_COPY)],
            sem,
        ).start()
    for c in range(n_copies):
        pltpu.make_async_copy(
            pat_v,
            out_hbm.at[pl.ds(base + c * ROWS_PER_COPY, ROWS_PER_COPY)],
            sem,
        ).wait()


def _tc_prep_kernel(alpha_ref, batch_ref, tgt_ref, pat_ref):
    tgt_ref[...] = batch_ref[:, N - 1:]
    a0 = alpha_ref[0, 0]
    a1 = alpha_ref[0, 1]
    a2 = alpha_ref[0, 2]
    s_full = (a0 + a1 + a2) * (1.0 / V)
    s_last = (a0 + a1) * (1.0 / V)
    n_preds = pat_ref.shape[1]
    j = lax.broadcasted_iota(jnp.int32, pat_ref.shape, 1)
    pat_ref[...] = jnp.where(j < n_preds - 1, s_full, s_last)


def kernel(batch, TEXT, alpha):
    B, bptt = batch.shape
    n_preds = bptt - (N - 1) + 1
    n_tgt = n_preds - 1
    rows_per_tile = B // 32

    targets, pattern = pl.pallas_call(
        _tc_prep_kernel,
        out_shape=[
            jax.ShapeDtypeStruct((B, n_tgt), batch.dtype),
            jax.ShapeDtypeStruct((ROWS_PER_COPY, n_preds, V), jnp.float32),
        ],
    )(alpha.reshape(1, 3), batch)

    mesh = plsc.VectorSubcoreMesh(core_axis_name="c", subcore_axis_name="s")
    sc_fill = functools.partial(
        pl.kernel,
        mesh=mesh,
        out_type=jax.ShapeDtypeStruct((B, n_preds, V), jnp.float32),
        scratch_types=[
            pltpu.VMEM((ROWS_PER_COPY, n_preds, V), jnp.float32),
            pltpu.SemaphoreType.DMA,
        ],
    )(functools.partial(_sc_fill_kernel, rows_per_tile))
    outputs = sc_fill(pattern)

    return outputs, targets


# speedup vs baseline: 4.8247x; 4.8247x over previous
import jax
import jax.numpy as jnp
from jax import lax
from jax.experimental import pallas as pl

V = 1000
N = 3


def _fill_kernel(alpha_ref, out_ref):
    a0 = alpha_ref[0, 0]
    a1 = alpha_ref[0, 1]
    a2 = alpha_ref[0, 2]
    s_full = (a0 + a1 + a2) * (1.0 / V)
    s_last = (a0 + a1) * (1.0 / V)
    n_preds = pl.num_programs(0)
    val = jnp.where(pl.program_id(0) < n_preds - 1, s_full, s_last)
    out_ref[...] = jnp.zeros(out_ref.shape, jnp.float32) + val


def _targets_kernel(batch_ref, tgt_ref):
    tgt_ref[...] = batch_ref[:, N - 1:]


def kernel(batch, TEXT, alpha):
    B, bptt = batch.shape
    n_preds = bptt - (N - 1) + 1
    n_tgt = n_preds - 1

    out_t = pl.pallas_call(
        _fill_kernel,
        grid=(n_preds,),
        in_specs=[pl.BlockSpec((1, 3), lambda i: (0, 0))],
        out_specs=pl.BlockSpec((1, V, B), lambda i: (i, 0, 0)),
        out_shape=jax.ShapeDtypeStruct((n_preds, V, B), jnp.float32),
    )(alpha.reshape(1, 3))
    outputs = jnp.transpose(out_t, (2, 0, 1))

    targets = pl.pallas_call(
        _targets_kernel,
        out_shape=jax.ShapeDtypeStruct((B, n_tgt), batch.dtype),
    )(batch)
    return outputs, targets
